# baseline (speedup 1.0000x reference)
import functools

import jax
import jax.numpy as jnp
from jax import lax
from jax.experimental import pallas as pl
from jax.experimental.pallas import tpu as pltpu
from jax.experimental.pallas import tpu_sc as plsc

R = 16384
C = 1000
K = int(R * 0.7)

NC = 2
NS = 16
NW = NC * NS
L = 16

R_SC = 8192
R_TC = R - R_SC
RPW = R_SC // NW
GPW = RPW // L
RT_PW = R_TC // NW
PCHUNK = 128

BR = 2048
NB_TC = R_TC // BR

_UNROLL = 25


def _sc_body(pflat_hbm, tgt_hbm,
             m_hbm, s_hbm, psc_hbm, ptc_hbm,
             xbuf, tvbuf, macc, sacc, pacc, tidx, fidx, pkbuf, sem):
    wid = lax.axis_index("s") * NC + lax.axis_index("c")
    base = wid * RPW
    lane = lax.broadcasted_iota(jnp.int32, (L,), 0)

    tc_row0 = R_SC + wid * RT_PW
    pltpu.sync_copy(tgt_hbm.at[pl.ds(tc_row0, RT_PW)], tidx)

    def mk_idx(g, _):
        tv = tidx[pl.ds(g * L, L)]
        rows = lane + (tc_row0 + g * L)
        fidx[pl.ds(g * L, L)] = rows * C + tv
        return 0

    lax.fori_loop(0, RT_PW // L, mk_idx, 0)
    for ch in range(RT_PW // PCHUNK):
        pltpu.async_copy(
            pflat_hbm.at[fidx.at[pl.ds(ch * PCHUNK, PCHUNK)]],
            pkbuf.at[pl.ds(ch * PCHUNK, PCHUNK)],
            sem,
        ).wait()
    pltpu.sync_copy(pkbuf, ptc_hbm.at[pl.ds(wid * RT_PW, RT_PW)])

    lanec = lane * C

    def group(g, _):
        row0 = base + g * L
        pltpu.sync_copy(pflat_hbm.at[pl.ds(row0 * C, L * C)], xbuf)
        pltpu.sync_copy(tgt_hbm.at[pl.ds(row0, L)], tvbuf)

        def p1(jo, m):
            for u in range(_UNROLL):
                j = jo * _UNROLL + u
                v = plsc.load_gather(xbuf, [lanec + j])
                m = jnp.maximum(m, v)
            return m

        m = lax.fori_loop(0, C // _UNROLL, p1,
                          jnp.full((L,), -jnp.inf, jnp.float32))

        def p2(jo, s):
            for u in range(_UNROLL):
                j = jo * _UNROLL + u
                v = plsc.load_gather(xbuf, [lanec + j])
                s = s + jnp.exp(v - m)
            return s

        s = lax.fori_loop(0, C // _UNROLL, p2, jnp.zeros((L,), jnp.float32))

        tv = tvbuf[...]
        pk = plsc.load_gather(xbuf, [lanec + tv])
        macc[pl.ds(g * L, L)] = m
        sacc[pl.ds(g * L, L)] = s
        pacc[pl.ds(g * L, L)] = pk
        return 0

    lax.fori_loop(0, GPW, group, 0)
    pltpu.sync_copy(macc, m_hbm.at[pl.ds(base, RPW)])
    pltpu.sync_copy(sacc, s_hbm.at[pl.ds(base, RPW)])
    pltpu.sync_copy(pacc, psc_hbm.at[pl.ds(base, RPW)])


_sc_ce = functools.partial(
    pl.kernel,
    out_type=[
        jax.ShapeDtypeStruct((R_SC,), jnp.float32),
        jax.ShapeDtypeStruct((R_SC,), jnp.float32),
        jax.ShapeDtypeStruct((R_SC,), jnp.float32),
        jax.ShapeDtypeStruct((R_TC,), jnp.float32),
    ],
    mesh=plsc.VectorSubcoreMesh(core_axis_name="c", subcore_axis_name="s"),
    compiler_params=pltpu.CompilerParams(needs_layout_passes=False),
    scratch_types=[
        pltpu.VMEM((L * C,), jnp.float32),
        pltpu.VMEM((L,), jnp.int32),
        pltpu.VMEM((RPW,), jnp.float32),
        pltpu.VMEM((RPW,), jnp.float32),
        pltpu.VMEM((RPW,), jnp.float32),
        pltpu.VMEM((RT_PW,), jnp.int32),
        pltpu.VMEM((RT_PW,), jnp.int32),
        pltpu.VMEM((RT_PW,), jnp.float32),
        pltpu.SemaphoreType.DMA,
    ],
)(_sc_body)


def _tc_lse_body(pred_ref, out_ref):
    x = pred_ref[...]
    m = jnp.max(x, axis=1)
    s = jnp.sum(jnp.exp(x - m[:, None]), axis=1)
    out_ref[0, 0, :] = m + jnp.log(s)


def _tc_lse(pred):
    out = pl.pallas_call(
        _tc_lse_body,
        grid=(NB_TC,),
        in_specs=[pl.BlockSpec((BR, C), lambda i: (i + R_SC // BR, 0))],
        out_specs=pl.BlockSpec((1, 1, BR), lambda i: (i, 0, 0)),
        out_shape=jax.ShapeDtypeStruct((NB_TC, 1, BR), jnp.float32),
    )(pred)
    return out.reshape(R_TC)


def _select_body(m_ref, s_ref, psc_ref, lse_ref, ptc_ref, out_ref):
    loss_sc = m_ref[...] + jnp.log(s_ref[...]) - psc_ref[...]
    loss_tc = lse_ref[...] - ptc_ref[...]
    vals = jnp.concatenate([loss_sc, loss_tc])
    u = lax.bitcast_convert_type(vals, jnp.uint32)
    sk = u ^ jnp.where(
        u >= jnp.uint32(0x80000000),
        jnp.uint32(0xFFFFFFFF),
        jnp.uint32(0x80000000),
    )

    def body(it, p):
        cand = p | (jnp.uint32(1) << (31 - it).astype(jnp.uint32))
        cnt = jnp.sum((sk >= cand).astype(jnp.int32))
        return jnp.where(cnt >= K, cand, p)

    p = lax.fori_loop(0, 32, body, jnp.uint32(0))

    gt = sk > p
    cnt_gt = jnp.sum(gt.astype(jnp.int32))
    sum_gt = jnp.sum(jnp.where(gt, vals, 0.0))
    orig = jnp.where(
        (p & jnp.uint32(0x80000000)) != jnp.uint32(0),
        p ^ jnp.uint32(0x80000000),
        ~p,
    )
    tau = lax.bitcast_convert_type(orig, jnp.float32)
    total = sum_gt + (K - cnt_gt).astype(jnp.float32) * tau
    out_ref[0, 0] = total / K


def _select(m_sc, s_sc, p_sc, lse_tc, p_tc):
    return pl.pallas_call(
        _select_body,
        out_specs=pl.BlockSpec(memory_space=pltpu.SMEM),
        out_shape=jax.ShapeDtypeStruct((1, 1), jnp.float32),
    )(m_sc, s_sc, p_sc, lse_tc, p_tc)


def kernel(pred, target):
    tgt = target.astype(jnp.int32)
    pred_flat = pred.reshape(R * C)
    m_sc, s_sc, p_sc, p_tc = _sc_ce(pred_flat, tgt)
    lse_tc = _tc_lse(pred)
    out = _select(m_sc, s_sc, p_sc, lse_tc, p_tc)
    return out[0, 0]
